# baseline (speedup 1.0000x reference)
import jax
import jax.numpy as jnp
from jax.experimental import pallas as pl
from jax.experimental.pallas import tpu as pltpu


def _copy_body(wpe_ref, out_ref):
    out_ref[...] = wpe_ref[...]


def kernel(x, wpe):
    B, S = x.shape
    R, D = wpe.shape
    BLK = 1024
    nblk = S // BLK
    out = pl.pallas_call(
        _copy_body,
        grid=(nblk, B),
        in_specs=[pl.BlockSpec((BLK, D), lambda i, b: (i, 0))],
        out_specs=pl.BlockSpec((BLK, D), lambda i, b: (b * nblk + i, 0)),
        out_shape=jax.ShapeDtypeStruct((B * S, D), wpe.dtype),
    )(wpe)
    return out.reshape(B, S, D)
